# baseline (speedup 1.0000x reference)
import jax
import jax.numpy as jnp
from jax.experimental import pallas as pl
from jax.experimental.pallas import tpu as pltpu

_N_STATES = 4
_HIDDEN = 25
_N_ACT = 2
_LANE_TILE = 524288
_CHUNK = 4096

_CONTRACT0 = (((0,), (0,)), ((), ()))


def _mlp_t_kernel(x_ref, w1_ref, b1_ref, w2_ref, b2_ref, q_ref, a_ref):
    lanes = x_ref.shape[1]
    ch = min(_CHUNK, lanes)
    nc = lanes // ch
    w1 = w1_ref[...]
    w2 = w2_ref[:, :_N_ACT]
    b1c = jnp.transpose(b1_ref[...])
    b2c = jnp.transpose(b2_ref[:, :_N_ACT])

    def dot1(c):
        xc = x_ref[:, c * ch:(c + 1) * ch]
        return jax.lax.dot_general(w1, xc, _CONTRACT0,
                                   preferred_element_type=jnp.float32)

    def dot2(h):
        hr = jnp.maximum(h + b1c, 0.0)
        return jax.lax.dot_general(w2, hr, _CONTRACT0,
                                   preferred_element_type=jnp.float32)

    def emit(c, q0):
        q = q0 + b2c
        q_ref[:, c * ch:(c + 1) * ch] = q
        a_ref[:, c * ch:(c + 1) * ch] = (q[1:2, :] > q[0:1, :]).astype(jnp.int32)

    hbuf = [None] * nc
    qbuf = [None] * nc
    for c in range(nc + 4):
        if c < nc:
            hbuf[c] = dot1(c)
        if 2 <= c < nc + 2:
            qbuf[c - 2] = dot2(hbuf[c - 2])
            hbuf[c - 2] = None
        if c >= 4:
            emit(c - 4, qbuf[c - 4])
            qbuf[c - 4] = None


def kernel(x, w1_t, b1_2d, w2_p, b2_p):
    B = x.shape[0]
    xt = x.T

    lane_tile = _LANE_TILE if B % _LANE_TILE == 0 else B

    q_t, a_t = pl.pallas_call(
        _mlp_t_kernel,
        grid=(B // lane_tile,),
        in_specs=[
            pl.BlockSpec((_N_STATES, lane_tile), lambda i: (0, i)),
            pl.BlockSpec((_N_STATES, _HIDDEN), lambda i: (0, 0)),
            pl.BlockSpec((1, _HIDDEN), lambda i: (0, 0)),
            pl.BlockSpec((_HIDDEN, 128), lambda i: (0, 0)),
            pl.BlockSpec((1, 128), lambda i: (0, 0)),
        ],
        out_specs=(
            pl.BlockSpec((_N_ACT, lane_tile), lambda i: (0, i)),
            pl.BlockSpec((1, lane_tile), lambda i: (0, i)),
        ),
        out_shape=(
            jax.ShapeDtypeStruct((_N_ACT, B), jnp.float32),
            jax.ShapeDtypeStruct((1, B), jnp.int32),
        ),
        compiler_params=pltpu.CompilerParams(
            dimension_semantics=("parallel",),
        ),
    )(xt, w1_t, b1_2d, w2_p, b2_p)

    return q_t.T, a_t.reshape(B)


# speedup vs baseline: 1.0158x; 1.0158x over previous
import jax
import jax.numpy as jnp
from jax.experimental import pallas as pl
from jax.experimental.pallas import tpu as pltpu

_N_STATES = 4
_HIDDEN = 25
_N_ACT = 2
_LANE_TILE = 262144
_CHUNK = 4096

_CONTRACT0 = (((0,), (0,)), ((), ()))


def _mlp_t_kernel(x_ref, w1_ref, b1_ref, w2_ref, b2_ref, q_ref, a_ref):
    lanes = x_ref.shape[1]
    ch = min(_CHUNK, lanes)
    nc = lanes // ch
    w1 = w1_ref[...]
    w2 = w2_ref[:, :_N_ACT]
    b1c = jnp.transpose(b1_ref[...])
    b2c = jnp.transpose(b2_ref[:, :_N_ACT])

    def dot1(c):
        xc = x_ref[:, c * ch:(c + 1) * ch]
        return jax.lax.dot_general(w1, xc, _CONTRACT0,
                                   preferred_element_type=jnp.float32)

    def dot2(h):
        hr = jnp.maximum(h + b1c, 0.0)
        return jax.lax.dot_general(w2, hr, _CONTRACT0,
                                   preferred_element_type=jnp.float32)

    def emit(c, q0):
        q = q0 + b2c
        q_ref[:, c * ch:(c + 1) * ch] = q
        a_ref[:, c * ch:(c + 1) * ch] = (q[1:2, :] > q[0:1, :]).astype(jnp.int32)

    hbuf = [None] * nc
    qbuf = [None] * nc
    for c in range(nc + 4):
        if c < nc:
            hbuf[c] = dot1(c)
        if 2 <= c < nc + 2:
            qbuf[c - 2] = dot2(hbuf[c - 2])
            hbuf[c - 2] = None
        if c >= 4:
            emit(c - 4, qbuf[c - 4])
            qbuf[c - 4] = None


def kernel(x, w1_t, b1_2d, w2_p, b2_p):
    B = x.shape[0]
    xt = x.T

    lane_tile = _LANE_TILE if B % _LANE_TILE == 0 else B

    q_t, a_t = pl.pallas_call(
        _mlp_t_kernel,
        grid=(B // lane_tile,),
        in_specs=[
            pl.BlockSpec((_N_STATES, lane_tile), lambda i: (0, i)),
            pl.BlockSpec((_N_STATES, _HIDDEN), lambda i: (0, 0)),
            pl.BlockSpec((1, _HIDDEN), lambda i: (0, 0)),
            pl.BlockSpec((_HIDDEN, 128), lambda i: (0, 0)),
            pl.BlockSpec((1, 128), lambda i: (0, 0)),
        ],
        out_specs=(
            pl.BlockSpec((_N_ACT, lane_tile), lambda i: (0, i)),
            pl.BlockSpec((1, lane_tile), lambda i: (0, i)),
        ),
        out_shape=(
            jax.ShapeDtypeStruct((_N_ACT, B), jnp.float32),
            jax.ShapeDtypeStruct((1, B), jnp.int32),
        ),
        compiler_params=pltpu.CompilerParams(
            dimension_semantics=("parallel",),
        ),
    )(xt, w1_t, b1_2d, w2_p, b2_p)

    return q_t.T, a_t.reshape(B)
